# baseline (speedup 1.0000x reference)
import jax
import jax.numpy as jnp
from jax import lax
from jax.experimental import pallas as pl
from jax.experimental.pallas import tpu as pltpu
from jax.experimental.pallas import tpu_sc as plsc

N = 10000
E = 320000
D = 128
NC = 2
NS = 16
NW = NC * NS
C = 128
K0 = 112
K1 = 46
KC = (K0 + K1) // 2
E_PAD = NS * (K0 + K1) * C
N_PAD = 10112
RPT = N_PAD // NS
CHUNKS = (C, C, C, C, RPT - 4 * C)

_MESH = plsc.VectorSubcoreMesh(core_axis_name="c", subcore_axis_name="s")
_OUT = jax.ShapeDtypeStruct((NC, N_PAD, D), jnp.float32)


def _zero_acc(zf, rows_v, acc, r0):
    pltpu.sync_copy(zf, rows_v)
    off = r0
    for ch in CHUNKS:
        pltpu.sync_copy(rows_v.at[pl.ds(0, ch)], acc.at[pl.ds(off, ch)])
        off += ch


def _drain_acc(acc, rows_v, out, cid, r0):
    off = r0
    for ch in CHUNKS:
        pltpu.sync_copy(acc.at[pl.ds(off, ch)], rows_v.at[pl.ds(0, ch)])
        pltpu.sync_copy(rows_v.at[pl.ds(0, ch)], out.at[cid, pl.ds(off, ch)])
        off += ch


def _feat_body(*refs):
        (feat, src1d, dst1d, zf, out, acc,
         src0, dst0, rows0, sem0, src1, dst1, rows1, sem1) = refs
        cid = lax.axis_index("c")
        sid = lax.axis_index("s")
        r0 = sid * RPT

        base = jnp.where(cid == 0, sid * K0, NS * K0 + sid * K1)
        half = jnp.where(cid == 0, K0 // 2, K1 // 2)

        _zero_acc(zf, rows0, acc, r0)
        plsc.subcore_barrier()

        def fill(b, src_v, dst_v, rows_v, sem):
            e = (base + b) * C
            pltpu.sync_copy(src1d.at[pl.ds(e, C)], src_v)
            pltpu.sync_copy(dst1d.at[pl.ds(e, C)], dst_v)
            pltpu.async_copy(feat.at[src_v], rows_v, sem)

        fill(0, src0, dst0, rows0, sem0)
        fill(1, src1, dst1, rows1, sem1)

        def pair(p, carry):
            pltpu.make_async_copy(feat.at[src0], rows0, sem0).wait()
            pltpu.sync_copy(rows0, acc.at[dst0], add=True)

            @pl.when(p < half - 1)
            def _():
                fill(2 * p + 2, src0, dst0, rows0, sem0)

            pltpu.make_async_copy(feat.at[src1], rows1, sem1).wait()
            pltpu.sync_copy(rows1, acc.at[dst1], add=True)

            @pl.when(p < half - 1)
            def _():
                fill(2 * p + 3, src1, dst1, rows1, sem1)

            return carry

        lax.fori_loop(0, half, pair, 0)

        plsc.subcore_barrier()
        _drain_acc(acc, rows0, out, cid, r0)


def _count_body(dst1d, zf, ones_h, out, acc, dst_v, rows_v, sem):
    cid = lax.axis_index("c")
    sid = lax.axis_index("s")
    wid = cid * NS + sid
    r0 = sid * RPT

    _zero_acc(zf, rows_v, acc, r0)
    plsc.subcore_barrier()
    pltpu.sync_copy(ones_h, rows_v)

    def step(j, carry):
        eoff = (wid * KC + j) * C
        pltpu.sync_copy(dst1d.at[pl.ds(eoff, C)], dst_v)
        pltpu.sync_copy(rows_v, acc.at[dst_v], add=True)
        return carry

    lax.fori_loop(0, KC, step, 0)

    plsc.subcore_barrier()
    _drain_acc(acc, rows_v, out, cid, r0)


_PIPE_SCRATCH = [
    pltpu.VMEM((C,), jnp.int32),
    pltpu.VMEM((C,), jnp.int32),
    pltpu.VMEM((C, D), jnp.float32),
    pltpu.SemaphoreType.DMA,
    pltpu.VMEM((C,), jnp.int32),
    pltpu.VMEM((C,), jnp.int32),
    pltpu.VMEM((C, D), jnp.float32),
    pltpu.SemaphoreType.DMA,
]

_sc_feat = pl.kernel(
    _feat_body,
    out_type=_OUT,
    mesh=_MESH,
    scratch_types=[pltpu.VMEM_SHARED((N_PAD, D), jnp.float32)] + _PIPE_SCRATCH,
)

_sc_count = pl.kernel(
    _count_body,
    out_type=_OUT,
    mesh=_MESH,
    scratch_types=[
        pltpu.VMEM_SHARED((N_PAD, D), jnp.float32),
        pltpu.VMEM((C,), jnp.int32),
        pltpu.VMEM((C, D), jnp.float32),
        pltpu.SemaphoreType.DMA,
    ],
)

BN = 1000


def _dense1_body(s1, c1, x, w1l, b1l, w1r, w2l, w2r, g, hr):
    cnt = c1[0][:, 0:1] + c1[1][:, 0:1]
    mean = (s1[0] + s1[1]) / jnp.maximum(cnt, 1.0)
    dims = (((1,), (1,)), ((), ()))
    h = (lax.dot_general(mean, w1l[...], dims, preferred_element_type=jnp.float32)
         + b1l[...]
         + lax.dot_general(x[...], w1r[...], dims, preferred_element_type=jnp.float32))
    h = jnp.maximum(h, 0.0)
    g[...] = lax.dot_general(h, w2l[...], dims, preferred_element_type=jnp.float32)
    hr[...] = lax.dot_general(h, w2r[...], dims, preferred_element_type=jnp.float32)


def _dense2_body(s2, c1, hr, b2l, out):
    cnt = c1[0][:, 0:1] + c1[1][:, 0:1]
    o = (s2[0] + s2[1]) / jnp.maximum(cnt, 1.0) + hr[...] + b2l[...]
    m = jnp.max(o, axis=1, keepdims=True)
    e = jnp.exp(o - m)
    out[...] = (o - m) - jnp.log(jnp.sum(e, axis=1, keepdims=True))


_dense1 = pl.pallas_call(
    _dense1_body,
    grid=(N // BN,),
    in_specs=[
        pl.BlockSpec((NC, BN, D), lambda i: (0, i, 0)),
        pl.BlockSpec((NC, BN, D), lambda i: (0, i, 0)),
        pl.BlockSpec((BN, D), lambda i: (i, 0)),
        pl.BlockSpec((256, D), lambda i: (0, 0)),
        pl.BlockSpec((1, 256), lambda i: (0, 0)),
        pl.BlockSpec((256, D), lambda i: (0, 0)),
        pl.BlockSpec((D, 256), lambda i: (0, 0)),
        pl.BlockSpec((D, 256), lambda i: (0, 0)),
    ],
    out_specs=[
        pl.BlockSpec((BN, D), lambda i: (i, 0)),
        pl.BlockSpec((BN, D), lambda i: (i, 0)),
    ],
    out_shape=[
        jax.ShapeDtypeStruct((N, D), jnp.float32),
        jax.ShapeDtypeStruct((N, D), jnp.float32),
    ],
)

_dense2 = pl.pallas_call(
    _dense2_body,
    grid=(N // BN,),
    in_specs=[
        pl.BlockSpec((NC, BN, D), lambda i: (0, i, 0)),
        pl.BlockSpec((NC, BN, D), lambda i: (0, i, 0)),
        pl.BlockSpec((BN, D), lambda i: (i, 0)),
        pl.BlockSpec((1, D), lambda i: (0, 0)),
    ],
    out_specs=pl.BlockSpec((BN, D), lambda i: (i, 0)),
    out_shape=jax.ShapeDtypeStruct((N, D), jnp.float32),
)


def kernel(x, edge_index, W1l, b1l, W1r, W2l, b2l, W2r):
    src = edge_index[0]
    dst = edge_index[1]
    pad = E_PAD - E
    src1d = jnp.concatenate([src, jnp.zeros((pad,), jnp.int32)])
    dst1d = jnp.concatenate([dst, jnp.full((pad,), N, jnp.int32)])
    zf = jnp.zeros((C, D), jnp.float32)
    ones = jnp.ones((C, D), jnp.float32)

    c1 = _sc_count(dst1d, zf, ones)
    s1 = _sc_feat(x, src1d, dst1d, zf)
    g, hr = _dense1(s1, c1, x, W1l, b1l.reshape(1, -1), W1r, W2l, W2r)
    s2 = _sc_feat(g, src1d, dst1d, zf)
    return _dense2(s2, c1, hr, b2l.reshape(1, -1))


# speedup vs baseline: 1.0353x; 1.0353x over previous
import jax
import jax.numpy as jnp
from jax import lax
from jax.experimental import pallas as pl
from jax.experimental.pallas import tpu as pltpu
from jax.experimental.pallas import tpu_sc as plsc

N = 10000
E = 320000
D = 128
NC = 2
NS = 16
NW = NC * NS
C = 128
K0 = 120
K1 = 38
KC = (K0 + K1) // 2
E_PAD = NS * (K0 + K1) * C
N_PAD = 10112
RPT = N_PAD // NS
CHUNKS = (C, C, C, C, RPT - 4 * C)

_MESH = plsc.VectorSubcoreMesh(core_axis_name="c", subcore_axis_name="s")
_OUT = jax.ShapeDtypeStruct((NC, N_PAD, D), jnp.float32)


def _zero_acc(zf, rows_v, acc, r0):
    pltpu.sync_copy(zf, rows_v)
    off = r0
    for ch in CHUNKS:
        pltpu.sync_copy(rows_v.at[pl.ds(0, ch)], acc.at[pl.ds(off, ch)])
        off += ch


def _drain_acc(acc, rows_v, out, cid, r0):
    off = r0
    for ch in CHUNKS:
        pltpu.sync_copy(acc.at[pl.ds(off, ch)], rows_v.at[pl.ds(0, ch)])
        pltpu.sync_copy(rows_v.at[pl.ds(0, ch)], out.at[cid, pl.ds(off, ch)])
        off += ch


def _feat_body(*refs):
        (feat, src1d, dst1d, zf, out, acc,
         src0, dst0, rows0, sem0, src1, dst1, rows1, sem1) = refs
        cid = lax.axis_index("c")
        sid = lax.axis_index("s")
        r0 = sid * RPT

        base = jnp.where(cid == 0, sid * K0, NS * K0 + sid * K1)
        half = jnp.where(cid == 0, K0 // 2, K1 // 2)

        _zero_acc(zf, rows0, acc, r0)
        plsc.subcore_barrier()

        def fill(b, src_v, dst_v, rows_v, sem):
            e = (base + b) * C
            pltpu.sync_copy(src1d.at[pl.ds(e, C)], src_v)
            pltpu.sync_copy(dst1d.at[pl.ds(e, C)], dst_v)
            pltpu.async_copy(feat.at[src_v], rows_v, sem)

        fill(0, src0, dst0, rows0, sem0)
        fill(1, src1, dst1, rows1, sem1)

        def pair(p, carry):
            pltpu.make_async_copy(feat.at[src0], rows0, sem0).wait()
            pltpu.sync_copy(rows0, acc.at[dst0], add=True)

            @pl.when(p < half - 1)
            def _():
                fill(2 * p + 2, src0, dst0, rows0, sem0)

            pltpu.make_async_copy(feat.at[src1], rows1, sem1).wait()
            pltpu.sync_copy(rows1, acc.at[dst1], add=True)

            @pl.when(p < half - 1)
            def _():
                fill(2 * p + 3, src1, dst1, rows1, sem1)

            return carry

        lax.fori_loop(0, half, pair, 0)

        plsc.subcore_barrier()
        _drain_acc(acc, rows0, out, cid, r0)


def _count_body(dst1d, zf, ones_h, out, acc, dst_v, rows_v, sem):
    cid = lax.axis_index("c")
    sid = lax.axis_index("s")
    wid = cid * NS + sid
    r0 = sid * RPT

    _zero_acc(zf, rows_v, acc, r0)
    plsc.subcore_barrier()
    pltpu.sync_copy(ones_h, rows_v)

    def step(j, carry):
        eoff = (wid * KC + j) * C
        pltpu.sync_copy(dst1d.at[pl.ds(eoff, C)], dst_v)
        pltpu.sync_copy(rows_v, acc.at[dst_v], add=True)
        return carry

    lax.fori_loop(0, KC, step, 0)

    plsc.subcore_barrier()
    _drain_acc(acc, rows_v, out, cid, r0)


_PIPE_SCRATCH = [
    pltpu.VMEM((C,), jnp.int32),
    pltpu.VMEM((C,), jnp.int32),
    pltpu.VMEM((C, D), jnp.float32),
    pltpu.SemaphoreType.DMA,
    pltpu.VMEM((C,), jnp.int32),
    pltpu.VMEM((C,), jnp.int32),
    pltpu.VMEM((C, D), jnp.float32),
    pltpu.SemaphoreType.DMA,
]

_sc_feat = pl.kernel(
    _feat_body,
    out_type=_OUT,
    mesh=_MESH,
    scratch_types=[pltpu.VMEM_SHARED((N_PAD, D), jnp.float32)] + _PIPE_SCRATCH,
)

_sc_count = pl.kernel(
    _count_body,
    out_type=_OUT,
    mesh=_MESH,
    scratch_types=[
        pltpu.VMEM_SHARED((N_PAD, D), jnp.float32),
        pltpu.VMEM((C,), jnp.int32),
        pltpu.VMEM((C, D), jnp.float32),
        pltpu.SemaphoreType.DMA,
    ],
)

BN = 1000


def _dense1_body(s1, c1, x, w1l, b1l, w1r, w2l, w2r, g, hr):
    cnt = c1[0][:, 0:1] + c1[1][:, 0:1]
    mean = (s1[0] + s1[1]) / jnp.maximum(cnt, 1.0)
    dims = (((1,), (1,)), ((), ()))
    h = (lax.dot_general(mean, w1l[...], dims, preferred_element_type=jnp.float32)
         + b1l[...]
         + lax.dot_general(x[...], w1r[...], dims, preferred_element_type=jnp.float32))
    h = jnp.maximum(h, 0.0)
    g[...] = lax.dot_general(h, w2l[...], dims, preferred_element_type=jnp.float32)
    hr[...] = lax.dot_general(h, w2r[...], dims, preferred_element_type=jnp.float32)


def _dense2_body(s2, c1, hr, b2l, out):
    cnt = c1[0][:, 0:1] + c1[1][:, 0:1]
    o = (s2[0] + s2[1]) / jnp.maximum(cnt, 1.0) + hr[...] + b2l[...]
    m = jnp.max(o, axis=1, keepdims=True)
    e = jnp.exp(o - m)
    out[...] = (o - m) - jnp.log(jnp.sum(e, axis=1, keepdims=True))


_dense1 = pl.pallas_call(
    _dense1_body,
    grid=(N // BN,),
    in_specs=[
        pl.BlockSpec((NC, BN, D), lambda i: (0, i, 0)),
        pl.BlockSpec((NC, BN, D), lambda i: (0, i, 0)),
        pl.BlockSpec((BN, D), lambda i: (i, 0)),
        pl.BlockSpec((256, D), lambda i: (0, 0)),
        pl.BlockSpec((1, 256), lambda i: (0, 0)),
        pl.BlockSpec((256, D), lambda i: (0, 0)),
        pl.BlockSpec((D, 256), lambda i: (0, 0)),
        pl.BlockSpec((D, 256), lambda i: (0, 0)),
    ],
    out_specs=[
        pl.BlockSpec((BN, D), lambda i: (i, 0)),
        pl.BlockSpec((BN, D), lambda i: (i, 0)),
    ],
    out_shape=[
        jax.ShapeDtypeStruct((N, D), jnp.float32),
        jax.ShapeDtypeStruct((N, D), jnp.float32),
    ],
)

_dense2 = pl.pallas_call(
    _dense2_body,
    grid=(N // BN,),
    in_specs=[
        pl.BlockSpec((NC, BN, D), lambda i: (0, i, 0)),
        pl.BlockSpec((NC, BN, D), lambda i: (0, i, 0)),
        pl.BlockSpec((BN, D), lambda i: (i, 0)),
        pl.BlockSpec((1, D), lambda i: (0, 0)),
    ],
    out_specs=pl.BlockSpec((BN, D), lambda i: (i, 0)),
    out_shape=jax.ShapeDtypeStruct((N, D), jnp.float32),
)


def kernel(x, edge_index, W1l, b1l, W1r, W2l, b2l, W2r):
    src = edge_index[0]
    dst = edge_index[1]
    pad = E_PAD - E
    src1d = jnp.concatenate([src, jnp.zeros((pad,), jnp.int32)])
    dst1d = jnp.concatenate([dst, jnp.full((pad,), N, jnp.int32)])
    zf = jnp.zeros((C, D), jnp.float32)
    ones = jnp.ones((C, D), jnp.float32)

    c1 = _sc_count(dst1d, zf, ones)
    s1 = _sc_feat(x, src1d, dst1d, zf)
    g, hr = _dense1(s1, c1, x, W1l, b1l.reshape(1, -1), W1r, W2l, W2r)
    s2 = _sc_feat(g, src1d, dst1d, zf)
    return _dense2(s2, c1, hr, b2l.reshape(1, -1))


# speedup vs baseline: 1.0386x; 1.0031x over previous
import jax
import jax.numpy as jnp
from jax import lax
from jax.experimental import pallas as pl
from jax.experimental.pallas import tpu as pltpu
from jax.experimental.pallas import tpu_sc as plsc

N = 10000
E = 320000
D = 128
NC = 2
NS = 16
NW = NC * NS
C = 128
K0 = 124
K1 = 34
KC = (K0 + K1) // 2
E_PAD = NS * (K0 + K1) * C
N_PAD = 10112
RPT = N_PAD // NS
CHUNKS = (C, C, C, C, RPT - 4 * C)

_MESH = plsc.VectorSubcoreMesh(core_axis_name="c", subcore_axis_name="s")
_OUT = jax.ShapeDtypeStruct((NC, N_PAD, D), jnp.float32)


def _zero_acc(zf, rows_v, acc, r0):
    pltpu.sync_copy(zf, rows_v)
    off = r0
    for ch in CHUNKS:
        pltpu.sync_copy(rows_v.at[pl.ds(0, ch)], acc.at[pl.ds(off, ch)])
        off += ch


def _drain_acc(acc, rows_v, out, cid, r0):
    off = r0
    for ch in CHUNKS:
        pltpu.sync_copy(acc.at[pl.ds(off, ch)], rows_v.at[pl.ds(0, ch)])
        pltpu.sync_copy(rows_v.at[pl.ds(0, ch)], out.at[cid, pl.ds(off, ch)])
        off += ch


def _feat_body(*refs):
        (feat, src1d, dst1d, zf, out, acc,
         src0, dst0, rows0, sem0, src1, dst1, rows1, sem1) = refs
        cid = lax.axis_index("c")
        sid = lax.axis_index("s")
        r0 = sid * RPT

        base = jnp.where(cid == 0, sid * K0, NS * K0 + sid * K1)
        half = jnp.where(cid == 0, K0 // 2, K1 // 2)

        _zero_acc(zf, rows0, acc, r0)
        plsc.subcore_barrier()

        def fill(b, src_v, dst_v, rows_v, sem):
            e = (base + b) * C
            pltpu.sync_copy(src1d.at[pl.ds(e, C)], src_v)
            pltpu.sync_copy(dst1d.at[pl.ds(e, C)], dst_v)
            pltpu.async_copy(feat.at[src_v], rows_v, sem)

        fill(0, src0, dst0, rows0, sem0)
        fill(1, src1, dst1, rows1, sem1)

        def pair(p, carry):
            pltpu.make_async_copy(feat.at[src0], rows0, sem0).wait()
            pltpu.sync_copy(rows0, acc.at[dst0], add=True)

            @pl.when(p < half - 1)
            def _():
                fill(2 * p + 2, src0, dst0, rows0, sem0)

            pltpu.make_async_copy(feat.at[src1], rows1, sem1).wait()
            pltpu.sync_copy(rows1, acc.at[dst1], add=True)

            @pl.when(p < half - 1)
            def _():
                fill(2 * p + 3, src1, dst1, rows1, sem1)

            return carry

        lax.fori_loop(0, half, pair, 0)

        plsc.subcore_barrier()
        _drain_acc(acc, rows0, out, cid, r0)


def _count_body(dst1d, zf, ones_h, out, acc, dst_v, rows_v, sem):
    cid = lax.axis_index("c")
    sid = lax.axis_index("s")
    wid = cid * NS + sid
    r0 = sid * RPT

    _zero_acc(zf, rows_v, acc, r0)
    plsc.subcore_barrier()
    pltpu.sync_copy(ones_h, rows_v)

    def step(j, carry):
        eoff = (wid * KC + j) * C
        pltpu.sync_copy(dst1d.at[pl.ds(eoff, C)], dst_v)
        pltpu.sync_copy(rows_v, acc.at[dst_v], add=True)
        return carry

    lax.fori_loop(0, KC, step, 0)

    plsc.subcore_barrier()
    _drain_acc(acc, rows_v, out, cid, r0)


_PIPE_SCRATCH = [
    pltpu.VMEM((C,), jnp.int32),
    pltpu.VMEM((C,), jnp.int32),
    pltpu.VMEM((C, D), jnp.float32),
    pltpu.SemaphoreType.DMA,
    pltpu.VMEM((C,), jnp.int32),
    pltpu.VMEM((C,), jnp.int32),
    pltpu.VMEM((C, D), jnp.float32),
    pltpu.SemaphoreType.DMA,
]

_sc_feat = pl.kernel(
    _feat_body,
    out_type=_OUT,
    mesh=_MESH,
    scratch_types=[pltpu.VMEM_SHARED((N_PAD, D), jnp.float32)] + _PIPE_SCRATCH,
)

_sc_count = pl.kernel(
    _count_body,
    out_type=_OUT,
    mesh=_MESH,
    scratch_types=[
        pltpu.VMEM_SHARED((N_PAD, D), jnp.float32),
        pltpu.VMEM((C,), jnp.int32),
        pltpu.VMEM((C, D), jnp.float32),
        pltpu.SemaphoreType.DMA,
    ],
)

BN = 1000


def _dense1_body(s1, c1, x, w1l, b1l, w1r, w2l, w2r, g, hr):
    cnt = c1[0][:, 0:1] + c1[1][:, 0:1]
    mean = (s1[0] + s1[1]) / jnp.maximum(cnt, 1.0)
    dims = (((1,), (1,)), ((), ()))
    h = (lax.dot_general(mean, w1l[...], dims, preferred_element_type=jnp.float32)
         + b1l[...]
         + lax.dot_general(x[...], w1r[...], dims, preferred_element_type=jnp.float32))
    h = jnp.maximum(h, 0.0)
    g[...] = lax.dot_general(h, w2l[...], dims, preferred_element_type=jnp.float32)
    hr[...] = lax.dot_general(h, w2r[...], dims, preferred_element_type=jnp.float32)


def _dense2_body(s2, c1, hr, b2l, out):
    cnt = c1[0][:, 0:1] + c1[1][:, 0:1]
    o = (s2[0] + s2[1]) / jnp.maximum(cnt, 1.0) + hr[...] + b2l[...]
    m = jnp.max(o, axis=1, keepdims=True)
    e = jnp.exp(o - m)
    out[...] = (o - m) - jnp.log(jnp.sum(e, axis=1, keepdims=True))


_dense1 = pl.pallas_call(
    _dense1_body,
    grid=(N // BN,),
    in_specs=[
        pl.BlockSpec((NC, BN, D), lambda i: (0, i, 0)),
        pl.BlockSpec((NC, BN, D), lambda i: (0, i, 0)),
        pl.BlockSpec((BN, D), lambda i: (i, 0)),
        pl.BlockSpec((256, D), lambda i: (0, 0)),
        pl.BlockSpec((1, 256), lambda i: (0, 0)),
        pl.BlockSpec((256, D), lambda i: (0, 0)),
        pl.BlockSpec((D, 256), lambda i: (0, 0)),
        pl.BlockSpec((D, 256), lambda i: (0, 0)),
    ],
    out_specs=[
        pl.BlockSpec((BN, D), lambda i: (i, 0)),
        pl.BlockSpec((BN, D), lambda i: (i, 0)),
    ],
    out_shape=[
        jax.ShapeDtypeStruct((N, D), jnp.float32),
        jax.ShapeDtypeStruct((N, D), jnp.float32),
    ],
)

_dense2 = pl.pallas_call(
    _dense2_body,
    grid=(N // BN,),
    in_specs=[
        pl.BlockSpec((NC, BN, D), lambda i: (0, i, 0)),
        pl.BlockSpec((NC, BN, D), lambda i: (0, i, 0)),
        pl.BlockSpec((BN, D), lambda i: (i, 0)),
        pl.BlockSpec((1, D), lambda i: (0, 0)),
    ],
    out_specs=pl.BlockSpec((BN, D), lambda i: (i, 0)),
    out_shape=jax.ShapeDtypeStruct((N, D), jnp.float32),
)


def kernel(x, edge_index, W1l, b1l, W1r, W2l, b2l, W2r):
    src = edge_index[0]
    dst = edge_index[1]
    pad = E_PAD - E
    src1d = jnp.concatenate([src, jnp.zeros((pad,), jnp.int32)])
    dst1d = jnp.concatenate([dst, jnp.full((pad,), N, jnp.int32)])
    zf = jnp.zeros((C, D), jnp.float32)
    ones = jnp.ones((C, D), jnp.float32)

    c1 = _sc_count(dst1d, zf, ones)
    s1 = _sc_feat(x, src1d, dst1d, zf)
    g, hr = _dense1(s1, c1, x, W1l, b1l.reshape(1, -1), W1r, W2l, W2r)
    s2 = _sc_feat(g, src1d, dst1d, zf)
    return _dense2(s2, c1, hr, b2l.reshape(1, -1))
